# Initial kernel scaffold; baseline (speedup 1.0000x reference)
import jax
import jax.numpy as jnp
from jax.experimental import pallas as pl


def kernel(pred, pos_assigned_gt_inds, target, pred2, target2):
    raise NotImplementedError("write your pallas kernel here")


# speedup vs baseline: 6.9837x; 6.9837x over previous
import functools
import math

import jax
import jax.numpy as jnp
from jax import lax
from jax.experimental import pallas as pl
from jax.experimental.pallas import tpu as pltpu
from jax.experimental.pallas import tpu_sc as plsc

L = 16
NTILES = 16

_LN2 = 0.6931471805599453
_SQRT2 = 1.4142135623730951
_EPS = 1e-6
_SIGMA = 0.9
_C1 = -math.log(1.0 - _SIGMA)


def _vlog(x):
    bits = plsc.bitcast(x, jnp.int32)
    e = lax.shift_right_logical(bits, 23) - 127
    m = plsc.bitcast(
        (bits & jnp.int32(0x007FFFFF)) | jnp.int32(0x3F800000), jnp.float32)
    big = m > _SQRT2
    m = jnp.where(big, 0.5 * m, m)
    ef = (e + jnp.where(big, 1, 0)).astype(jnp.float32)
    s = (m - 1.0) / (m + 1.0)
    z = s * s
    p = 1.0 + z * (1.0 / 3.0 + z * (0.2 + z * (1.0 / 7.0 + z * (1.0 / 9.0))))
    return 2.0 * s * p + ef * _LN2


def _smooth_l1(d):
    ad = jnp.abs(d)
    return jnp.where(ad < 1.0, 0.5 * ad * ad, ad - 0.5)


def _sc_rep_loss(cols_flat, predT2, indsF, targT2, B, P, G, N, NPAD):
    CHUNK = NPAD // NTILES
    tiles_per_img = NTILES // B
    preds_per_tile = P // tiles_per_img
    gts_per_tile = G // tiles_per_img

    mesh = plsc.VectorSubcoreMesh(
        core_axis_name="c", subcore_axis_name="s", num_cores=1)

    @functools.partial(
        pl.kernel,
        out_type=jax.ShapeDtypeStruct((L,), jnp.float32),
        mesh=mesh,
        compiler_params=pltpu.CompilerParams(
            needs_layout_passes=False, use_tc_tiling_on_sc=False),
        scratch_types=[
            pltpu.VMEM((8, CHUNK), jnp.float32),
            pltpu.VMEM((4, P), jnp.float32),
            pltpu.VMEM((P,), jnp.int32),
            pltpu.VMEM((4, G), jnp.float32),
            pltpu.VMEM((G,), jnp.float32),
            pltpu.VMEM((L,), jnp.float32),
            pltpu.VMEM_SHARED((NTILES, L), jnp.float32),
            pltpu.VMEM((NTILES, L), jnp.float32),
            pltpu.VMEM((L,), jnp.float32),
            pltpu.SemaphoreType.DMA,
        ],
    )
    def run(cols_hbm, pred_hbm, inds_hbm, targ_hbm, out_hbm,
            colsv, predv, indsv, targv, gareav, partv, sharedp, allpv, outv,
            sem):
        wid = lax.axis_index("s")
        img = wid // tiles_per_img
        q = wid % tiles_per_img

        cps = []
        for c in range(8):
            cps.append(pltpu.async_copy(
                cols_hbm.at[pl.ds(c * NPAD + wid * CHUNK, CHUNK)],
                colsv.at[c], sem))
        cps.append(pltpu.async_copy(
            pred_hbm.at[pl.ds(img * 4, 4)], predv, sem))
        cps.append(pltpu.async_copy(
            inds_hbm.at[pl.ds(img * P, P)], indsv, sem))
        cps.append(pltpu.async_copy(
            targ_hbm.at[pl.ds(img * 4, 4)], targv, sem))
        for cp in cps:
            cp.wait()

        zeros = jnp.zeros((L,), jnp.float32)
        ones = jnp.ones((L,), jnp.float32)

        def iou_step(k, acc):
            o = k * L
            px1 = colsv[0, pl.ds(o, L)]
            py1 = colsv[1, pl.ds(o, L)]
            px2 = colsv[2, pl.ds(o, L)]
            py2 = colsv[3, pl.ds(o, L)]
            tx1 = colsv[4, pl.ds(o, L)]
            ty1 = colsv[5, pl.ds(o, L)]
            tx2 = colsv[6, pl.ds(o, L)]
            ty2 = colsv[7, pl.ds(o, L)]
            w = jnp.maximum(jnp.minimum(px2, tx2) - jnp.maximum(px1, tx1), 0.0)
            h = jnp.maximum(jnp.minimum(py2, ty2) - jnp.maximum(py1, ty1), 0.0)
            ov = w * h
            ap = (px2 - px1) * (py2 - py1)
            ag = (tx2 - tx1) * (ty2 - ty1)
            union = jnp.maximum(ap + ag - ov, _EPS)
            iou = jnp.maximum(ov / union, _EPS)
            return acc - _vlog(iou)

        iou_acc = lax.fori_loop(0, CHUNK // L, iou_step, zeros, unroll=2)
        iou_s = jnp.sum(iou_acc)

        ngc = G // L
        for gc in range(ngc):
            gareav[pl.ds(gc * L, L)] = (
                (targv[2, pl.ds(gc * L, L)] - targv[0, pl.ds(gc * L, L)])
                * (targv[3, pl.ds(gc * L, L)] - targv[1, pl.ds(gc * L, L)]))
        row = [jnp.full((L,), c, jnp.int32) for c in range(4)]

        NCH = 4
        rep_sv = zeros
        rep_nv = zeros
        for half in range(preds_per_tile // (NCH * L)):
            pdata = []
            for kc in range(NCH):
                base = q * preds_per_tile + (half * NCH + kc) * L
                pdata.append((predv[0, pl.ds(base, L)],
                              predv[1, pl.ds(base, L)],
                              predv[2, pl.ds(base, L)],
                              predv[3, pl.ds(base, L)],
                              indsv[pl.ds(base, L)]))

            def gstep(g, carry):
                bests, garbs = carry
                gidx = jnp.full((L,), g, jnp.int32)
                tx1 = plsc.load_gather(targv, [row[0], gidx])
                ty1 = plsc.load_gather(targv, [row[1], gidx])
                tx2 = plsc.load_gather(targv, [row[2], gidx])
                ty2 = plsc.load_gather(targv, [row[3], gidx])
                ga = plsc.load_gather(gareav, [gidx])
                nb, ng = [], []
                for kc in range(NCH):
                    px1, py1, px2, py2, pind = pdata[kc]
                    iw = jnp.maximum(
                        jnp.minimum(px2, tx2) - jnp.maximum(px1, tx1), 0.0)
                    ih = jnp.maximum(
                        jnp.minimum(py2, ty2) - jnp.maximum(py1, ty1), 0.0)
                    ov = jnp.where(pind == gidx, 0.0, iw * ih)
                    upd = ov > bests[kc]
                    nb.append(jnp.where(upd, ov, bests[kc]))
                    ng.append(jnp.where(upd, ga, garbs[kc]))
                return tuple(nb), tuple(ng)

            bests, garbs = lax.fori_loop(
                0, G, gstep, ((zeros,) * NCH, (ones,) * NCH))
            for kc in range(NCH):
                best = bests[kc]
                valid = best > 0.0
                iog = best / garbs[kc]
                one_m = jnp.maximum(1.0 - iog, _EPS)
                sml = jnp.where(iog > _SIGMA,
                                (iog - _SIGMA) * (1.0 / (1.0 - _SIGMA)) + _C1,
                                -_vlog(one_m))
                rep_sv = rep_sv + jnp.where(valid, sml, 0.0)
                rep_nv = rep_nv + jnp.where(valid, 1.0, 0.0)
        rep_s = jnp.sum(rep_sv)
        rep_n = jnp.sum(rep_nv)

        gtid = q * gts_per_tile + lax.broadcasted_iota(jnp.int32, (L,), 0)

        def pstep(p, carry):
            cnt, s1, s2, s3, s4 = carry
            pidx = jnp.full((L,), p, jnp.int32)
            ind_b = plsc.load_gather(indsv, [pidx])
            b1 = plsc.load_gather(predv, [row[0], pidx])
            b2 = plsc.load_gather(predv, [row[1], pidx])
            b3 = plsc.load_gather(predv, [row[2], pidx])
            b4 = plsc.load_gather(predv, [row[3], pidx])
            eq = gtid == ind_b
            cnt = cnt + jnp.where(eq, 1.0, 0.0)
            s1 = s1 + jnp.where(eq, b1, 0.0)
            s2 = s2 + jnp.where(eq, b2, 0.0)
            s3 = s3 + jnp.where(eq, b3, 0.0)
            s4 = s4 + jnp.where(eq, b4, 0.0)
            return cnt, s1, s2, s3, s4

        cnt, s1, s2, s3, s4 = lax.fori_loop(
            0, P, pstep, (zeros, zeros, zeros, zeros, zeros), unroll=4)
        cmax = jnp.maximum(cnt, 1.0)
        goff = q * gts_per_tile
        sl = (_smooth_l1(targv[0, pl.ds(goff, L)] - s1 / cmax)
              + _smooth_l1(targv[1, pl.ds(goff, L)] - s2 / cmax)
              + _smooth_l1(targv[2, pl.ds(goff, L)] - s3 / cmax)
              + _smooth_l1(targv[3, pl.ds(goff, L)] - s4 / cmax)) * 0.25
        gm = cnt > 1.0
        com_s = jnp.sum(jnp.where(gm, sl, 0.0))
        com_n = jnp.sum(jnp.where(gm, 1.0, 0.0))

        iv = lax.broadcasted_iota(jnp.int32, (L,), 0)
        pvec = (jnp.where(iv == 0, iou_s, 0.0)
                + jnp.where(iv == 1, rep_s, 0.0)
                + jnp.where(iv == 2, rep_n, 0.0)
                + jnp.where(iv == 3, com_s, 0.0)
                + jnp.where(iv == 4, com_n, 0.0))
        partv[...] = pvec
        pltpu.sync_copy(partv, sharedp.at[wid])
        plsc.subcore_barrier()

        @pl.when(wid == 0)
        def _finalize():
            pltpu.sync_copy(sharedp, allpv)
            acc = zeros
            for i in range(NTILES):
                acc = acc + allpv[i]
            t_iou = jnp.broadcast_to(acc[0], (L,))
            t_rep_s = jnp.broadcast_to(acc[1], (L,))
            t_rep_n = jnp.broadcast_to(acc[2], (L,))
            t_com_s = jnp.broadcast_to(acc[3], (L,))
            t_com_n = jnp.broadcast_to(acc[4], (L,))
            rep = jnp.where(t_rep_n > 0.0,
                            10.0 * t_rep_s / jnp.maximum(t_rep_n, 1.0), 0.0)
            com = jnp.where(t_com_n > 0.0,
                            10.0 * t_com_s / jnp.maximum(t_com_n, 1.0), 0.0)
            total = t_iou * (1.0 / N) + rep + com
            outv[...] = jnp.where(iv == 0, total, 0.0)
            pltpu.sync_copy(outv, out_hbm)

    return run(cols_flat, predT2, indsF, targT2)


def kernel(pred, pos_assigned_gt_inds, target, pred2, target2):
    B, P, _ = pred.shape
    G = target.shape[1]
    N = pred2.shape[0]
    NPAD = -(-N // (NTILES * L)) * (NTILES * L)

    cols = jnp.concatenate([pred2.T, target2.T], axis=0)
    if NPAD > N:
        padcol = jnp.array([0, 0, 1, 1, 0, 0, 1, 1], jnp.float32)[:, None]
        cols = jnp.concatenate(
            [cols, jnp.broadcast_to(padcol, (8, NPAD - N))], axis=1)
    cols_flat = cols.reshape(8 * NPAD)

    predT2 = pred.transpose(0, 2, 1).reshape(B * 4, P)
    targT2 = target.transpose(0, 2, 1).reshape(B * 4, G)
    indsF = pos_assigned_gt_inds.astype(jnp.int32).reshape(B * P)

    out = _sc_rep_loss(cols_flat, predT2, indsF, targT2, B, P, G, N, NPAD)
    return out[0]


# speedup vs baseline: 7.2220x; 1.0341x over previous
import functools
import math

import jax
import jax.numpy as jnp
from jax import lax
from jax.experimental import pallas as pl
from jax.experimental.pallas import tpu as pltpu
from jax.experimental.pallas import tpu_sc as plsc

L = 16
NTILES = 16

_LN2 = 0.6931471805599453
_SQRT2 = 1.4142135623730951
_EPS = 1e-6
_SIGMA = 0.9
_C1 = -math.log(1.0 - _SIGMA)


def _vlog(x):
    bits = plsc.bitcast(x, jnp.int32)
    e = lax.shift_right_logical(bits, 23) - 127
    m = plsc.bitcast(
        (bits & jnp.int32(0x007FFFFF)) | jnp.int32(0x3F800000), jnp.float32)
    big = m > _SQRT2
    m = jnp.where(big, 0.5 * m, m)
    ef = (e + jnp.where(big, 1, 0)).astype(jnp.float32)
    s = (m - 1.0) / (m + 1.0)
    z = s * s
    p = 1.0 + z * (1.0 / 3.0 + z * (0.2 + z * (1.0 / 7.0 + z * (1.0 / 9.0))))
    return 2.0 * s * p + ef * _LN2


def _smooth_l1(d):
    ad = jnp.abs(d)
    return jnp.where(ad < 1.0, 0.5 * ad * ad, ad - 0.5)


def _sc_rep_loss(cols_flat, predT2, indsF, targT2, B, P, G, N, NPAD):
    CHUNK = NPAD // NTILES
    tiles_per_img = NTILES // B
    preds_per_tile = P // tiles_per_img
    gts_per_tile = G // tiles_per_img

    mesh = plsc.VectorSubcoreMesh(
        core_axis_name="c", subcore_axis_name="s", num_cores=1)

    @functools.partial(
        pl.kernel,
        out_type=jax.ShapeDtypeStruct((L,), jnp.float32),
        mesh=mesh,
        compiler_params=pltpu.CompilerParams(
            needs_layout_passes=False, use_tc_tiling_on_sc=False),
        scratch_types=[
            pltpu.VMEM((8, CHUNK), jnp.float32),
            pltpu.VMEM((4, P), jnp.float32),
            pltpu.VMEM((P,), jnp.int32),
            pltpu.VMEM((4, G), jnp.float32),
            pltpu.VMEM((G,), jnp.float32),
            pltpu.VMEM((5, G), jnp.float32),
            pltpu.VMEM((L,), jnp.float32),
            pltpu.VMEM_SHARED((NTILES, L), jnp.float32),
            pltpu.VMEM((NTILES, L), jnp.float32),
            pltpu.VMEM((L,), jnp.float32),
            pltpu.SemaphoreType.DMA,
            pltpu.SemaphoreType.DMA,
        ],
    )
    def run(cols_hbm, pred_hbm, inds_hbm, targ_hbm, out_hbm,
            colsv, predv, indsv, targv, gareav, histv, partv, sharedp, allpv,
            outv, sem, sem2):
        wid = lax.axis_index("s")
        img = wid // tiles_per_img
        q = wid % tiles_per_img

        cols_cps = []
        for c in range(8):
            cols_cps.append(pltpu.async_copy(
                cols_hbm.at[pl.ds(c * NPAD + wid * CHUNK, CHUNK)],
                colsv.at[c], sem))
        small_cps = [
            pltpu.async_copy(pred_hbm.at[pl.ds(img * 4, 4)], predv, sem2),
            pltpu.async_copy(inds_hbm.at[pl.ds(img * P, P)], indsv, sem2),
            pltpu.async_copy(targ_hbm.at[pl.ds(img * 4, 4)], targv, sem2),
        ]
        for cp in small_cps:
            cp.wait()

        zeros = jnp.zeros((L,), jnp.float32)
        ones = jnp.ones((L,), jnp.float32)

        ngc = G // L
        for gc in range(ngc):
            gareav[pl.ds(gc * L, L)] = (
                (targv[2, pl.ds(gc * L, L)] - targv[0, pl.ds(gc * L, L)])
                * (targv[3, pl.ds(gc * L, L)] - targv[1, pl.ds(gc * L, L)]))
        row = [jnp.full((L,), c, jnp.int32) for c in range(4)]

        NCH = 4
        rep_sv = zeros
        rep_nv = zeros
        for half in range(preds_per_tile // (NCH * L)):
            pdata = []
            for kc in range(NCH):
                base = q * preds_per_tile + (half * NCH + kc) * L
                pdata.append((predv[0, pl.ds(base, L)],
                              predv[1, pl.ds(base, L)],
                              predv[2, pl.ds(base, L)],
                              predv[3, pl.ds(base, L)],
                              indsv[pl.ds(base, L)]))

            def gstep(g, carry):
                bests, garbs = carry
                gidx = jnp.full((L,), g, jnp.int32)
                tx1 = plsc.load_gather(targv, [row[0], gidx])
                ty1 = plsc.load_gather(targv, [row[1], gidx])
                tx2 = plsc.load_gather(targv, [row[2], gidx])
                ty2 = plsc.load_gather(targv, [row[3], gidx])
                ga = plsc.load_gather(gareav, [gidx])
                nb, ng = [], []
                for kc in range(NCH):
                    px1, py1, px2, py2, pind = pdata[kc]
                    iw = jnp.maximum(
                        jnp.minimum(px2, tx2) - jnp.maximum(px1, tx1), 0.0)
                    ih = jnp.maximum(
                        jnp.minimum(py2, ty2) - jnp.maximum(py1, ty1), 0.0)
                    ov = jnp.where(pind == gidx, 0.0, iw * ih)
                    upd = ov > bests[kc]
                    nb.append(jnp.where(upd, ov, bests[kc]))
                    ng.append(jnp.where(upd, ga, garbs[kc]))
                return tuple(nb), tuple(ng)

            bests, garbs = lax.fori_loop(
                0, G, gstep, ((zeros,) * NCH, (ones,) * NCH))
            for kc in range(NCH):
                best = bests[kc]
                valid = best > 0.0
                iog = best / garbs[kc]
                one_m = jnp.maximum(1.0 - iog, _EPS)
                sml = jnp.where(iog > _SIGMA,
                                (iog - _SIGMA) * (1.0 / (1.0 - _SIGMA)) + _C1,
                                -_vlog(one_m))
                rep_sv = rep_sv + jnp.where(valid, sml, 0.0)
                rep_nv = rep_nv + jnp.where(valid, 1.0, 0.0)
        rep_s = jnp.sum(rep_sv)
        rep_n = jnp.sum(rep_nv)

        gtid = q * gts_per_tile + lax.broadcasted_iota(jnp.int32, (L,), 0)

        for r in range(5):
            for c4 in range(G // L):
                histv[r, pl.ds(c4 * L, L)] = zeros
        row5 = jnp.full((L,), 4, jnp.int32)

        def pstep(kc, _):
            o = kc * L
            indv = indsv[pl.ds(o, L)]
            plsc.addupdate_scatter(histv, [row[0], indv], ones)
            plsc.addupdate_scatter(histv, [row[1], indv],
                                   predv[0, pl.ds(o, L)])
            plsc.addupdate_scatter(histv, [row[2], indv],
                                   predv[1, pl.ds(o, L)])
            plsc.addupdate_scatter(histv, [row[3], indv],
                                   predv[2, pl.ds(o, L)])
            plsc.addupdate_scatter(histv, [row5, indv],
                                   predv[3, pl.ds(o, L)])
            return 0

        lax.fori_loop(0, P // L, pstep, 0)
        goff0 = q * gts_per_tile
        cnt = histv[0, pl.ds(goff0, L)]
        s1 = histv[1, pl.ds(goff0, L)]
        s2 = histv[2, pl.ds(goff0, L)]
        s3 = histv[3, pl.ds(goff0, L)]
        s4 = histv[4, pl.ds(goff0, L)]
        cmax = jnp.maximum(cnt, 1.0)
        goff = q * gts_per_tile
        sl = (_smooth_l1(targv[0, pl.ds(goff, L)] - s1 / cmax)
              + _smooth_l1(targv[1, pl.ds(goff, L)] - s2 / cmax)
              + _smooth_l1(targv[2, pl.ds(goff, L)] - s3 / cmax)
              + _smooth_l1(targv[3, pl.ds(goff, L)] - s4 / cmax)) * 0.25
        gm = cnt > 1.0
        com_s = jnp.sum(jnp.where(gm, sl, 0.0))
        com_n = jnp.sum(jnp.where(gm, 1.0, 0.0))

        for cp in cols_cps:
            cp.wait()

        def iou_step(k, acc):
            o = k * L
            px1 = colsv[0, pl.ds(o, L)]
            py1 = colsv[1, pl.ds(o, L)]
            px2 = colsv[2, pl.ds(o, L)]
            py2 = colsv[3, pl.ds(o, L)]
            tx1 = colsv[4, pl.ds(o, L)]
            ty1 = colsv[5, pl.ds(o, L)]
            tx2 = colsv[6, pl.ds(o, L)]
            ty2 = colsv[7, pl.ds(o, L)]
            w = jnp.maximum(jnp.minimum(px2, tx2) - jnp.maximum(px1, tx1), 0.0)
            h = jnp.maximum(jnp.minimum(py2, ty2) - jnp.maximum(py1, ty1), 0.0)
            ov = w * h
            ap = (px2 - px1) * (py2 - py1)
            ag = (tx2 - tx1) * (ty2 - ty1)
            union = jnp.maximum(ap + ag - ov, _EPS)
            iou = jnp.maximum(ov / union, _EPS)
            return acc - _vlog(iou)

        iou_acc = lax.fori_loop(0, CHUNK // L, iou_step, zeros, unroll=2)
        iou_s = jnp.sum(iou_acc)

        iv = lax.broadcasted_iota(jnp.int32, (L,), 0)
        pvec = (jnp.where(iv == 0, iou_s, 0.0)
                + jnp.where(iv == 1, rep_s, 0.0)
                + jnp.where(iv == 2, rep_n, 0.0)
                + jnp.where(iv == 3, com_s, 0.0)
                + jnp.where(iv == 4, com_n, 0.0))
        partv[...] = pvec
        pltpu.sync_copy(partv, sharedp.at[wid])
        plsc.subcore_barrier()

        @pl.when(wid == 0)
        def _finalize():
            pltpu.sync_copy(sharedp, allpv)
            acc = zeros
            for i in range(NTILES):
                acc = acc + allpv[i]
            t_iou = jnp.broadcast_to(acc[0], (L,))
            t_rep_s = jnp.broadcast_to(acc[1], (L,))
            t_rep_n = jnp.broadcast_to(acc[2], (L,))
            t_com_s = jnp.broadcast_to(acc[3], (L,))
            t_com_n = jnp.broadcast_to(acc[4], (L,))
            rep = jnp.where(t_rep_n > 0.0,
                            10.0 * t_rep_s / jnp.maximum(t_rep_n, 1.0), 0.0)
            com = jnp.where(t_com_n > 0.0,
                            10.0 * t_com_s / jnp.maximum(t_com_n, 1.0), 0.0)
            total = t_iou * (1.0 / N) + rep + com
            outv[...] = jnp.where(iv == 0, total, 0.0)
            pltpu.sync_copy(outv, out_hbm)

    return run(cols_flat, predT2, indsF, targT2)


def kernel(pred, pos_assigned_gt_inds, target, pred2, target2):
    B, P, _ = pred.shape
    G = target.shape[1]
    N = pred2.shape[0]
    NPAD = -(-N // (NTILES * L)) * (NTILES * L)

    cols = jnp.concatenate([pred2.T, target2.T], axis=0)
    if NPAD > N:
        padcol = jnp.array([0, 0, 1, 1, 0, 0, 1, 1], jnp.float32)[:, None]
        cols = jnp.concatenate(
            [cols, jnp.broadcast_to(padcol, (8, NPAD - N))], axis=1)
    cols_flat = cols.reshape(8 * NPAD)

    predT2 = pred.transpose(0, 2, 1).reshape(B * 4, P)
    targT2 = target.transpose(0, 2, 1).reshape(B * 4, G)
    indsF = pos_assigned_gt_inds.astype(jnp.int32).reshape(B * P)

    out = _sc_rep_loss(cols_flat, predT2, indsF, targT2, B, P, G, N, NPAD)
    return out[0]
